# baseline (speedup 1.0000x reference)
import functools

import jax
import jax.numpy as jnp
from jax import lax
from jax.experimental import pallas as pl
from jax.experimental.pallas import tpu as pltpu
from jax.experimental.pallas import tpu_sc as plsc

BATCH = 16384
EMBED = 64

_info = plsc.get_sparse_core_info()
NC, NS, L = _info.num_cores, _info.num_subcores, _info.num_lanes
NW = NC * NS
B_PER_W = BATCH // NW
IDXW = 128
NCHUNK = B_PER_W // IDXW
GROUPS = B_PER_W // L


def _sc_kernel(users_hbm, items_hbm, ut_hbm, it_hbm, out_hbm,
               uidx_v, iidx_v, urows_v, irows_v, out_v, sem):
    wid = lax.axis_index("s") * NC + lax.axis_index("c")
    base = wid * B_PER_W

    pltpu.sync_copy(users_hbm.at[wid], uidx_v)
    pltpu.sync_copy(items_hbm.at[wid], iidx_v)

    cps = []
    for j in range(NCHUNK):
        dst = urows_v.at[pl.ds(j * IDXW, IDXW)]
        cps.append(pltpu.async_copy(ut_hbm.at[uidx_v.at[j]], dst, sem))
        dst = irows_v.at[pl.ds(j * IDXW, IDXW)]
        cps.append(pltpu.async_copy(it_hbm.at[iidx_v.at[j]], dst, sem))
    for cp in cps:
        cp.wait()

    iota = lax.iota(jnp.int32, L)

    def group_body(g, carry):
        row = g * L + iota
        acc = jnp.zeros((L,), jnp.float32)
        for d in range(EMBED):
            col = lax.bitwise_and(iota + d, EMBED - 1)
            u = plsc.load_gather(urows_v, [row, col])
            v = plsc.load_gather(irows_v, [row, col])
            acc = acc + u * v
        out_v[pl.ds(g * L, L)] = acc
        return carry

    lax.fori_loop(0, GROUPS, group_body, 0)

    pltpu.sync_copy(out_v, out_hbm.at[pl.ds(base, B_PER_W)])


def kernel(users, items, user_table, item_table):
    users_3d = users.reshape(NW, NCHUNK, IDXW)
    items_3d = items.reshape(NW, NCHUNK, IDXW)

    run = functools.partial(
        pl.kernel,
        mesh=plsc.VectorSubcoreMesh(core_axis_name="c", subcore_axis_name="s"),
        out_type=jax.ShapeDtypeStruct((BATCH,), jnp.float32),
        scratch_types=[
            pltpu.VMEM((NCHUNK, IDXW), jnp.int32),
            pltpu.VMEM((NCHUNK, IDXW), jnp.int32),
            pltpu.VMEM((B_PER_W, EMBED), jnp.float32),
            pltpu.VMEM((B_PER_W, EMBED), jnp.float32),
            pltpu.VMEM((B_PER_W,), jnp.float32),
            pltpu.SemaphoreType.DMA,
        ],
        compiler_params=pltpu.CompilerParams(
            needs_layout_passes=False, use_tc_tiling_on_sc=False
        ),
    )(_sc_kernel)
    return run(users_3d, items_3d, user_table, item_table)
